# baseline (speedup 1.0000x reference)
import functools
import jax
import jax.numpy as jnp
from jax import lax
from jax.experimental import pallas as pl
from jax.experimental.pallas import tpu as pltpu
from jax.experimental.pallas import tpu_sc as plsc

_SPB = 4

_NC, _NS = 2, 16
_NW = _NC * _NS
_CH = 128


def _body_a(enc_ref, dec_ref, Wpre_ref, bpre_ref, cbT_ref, cb_ref, Wpost_ref,
            bpost_ref, out_ref, codes_ref, cm_ref, kl_ref, cbW_out_ref,
            cbW_ref, cbT2_ref, c2_ref, Wpost_bf_ref):
    T, D = enc_ref.shape[1], enc_ref.shape[2]
    K = cb_ref.shape[0]
    b = pl.program_id(0)

    @pl.when(b == 0)
    def _init():
        cb = cb_ref[...]
        Wpost = Wpost_ref[...]
        cbW_ref[...] = jnp.dot(cb, Wpost,
                               preferred_element_type=jnp.float32)
        cbW_out_ref[...] = cbW_ref[...]
        Wpost_bf_ref[...] = Wpost.astype(jnp.bfloat16)
        cbT = cbT_ref[...]
        cbT2_ref[...] = cbT * -2.0
        c2_ref[...] = jnp.sum(cbT * cbT, axis=0, keepdims=True)

    iota = jax.lax.broadcasted_iota(
        jnp.int32, (T, K), 1).astype(jnp.float32)
    ones_row = jnp.ones((8, T), dtype=jnp.bfloat16)
    for i in range(_SPB):
        e = enc_ref[i]
        d = dec_ref[i]
        r = jnp.dot(e - d, Wpre_ref[...],
                    preferred_element_type=jnp.float32) + bpre_ref[...]
        scores = jnp.dot(r, cbT2_ref[...],
                         preferred_element_type=jnp.float32) + c2_ref[...]
        m = jnp.min(scores, axis=1)
        codes = jnp.min(jnp.where(scores <= m[:, None], iota, float(K)),
                        axis=1)
        onehot = jnp.where(iota == codes[:, None], 1.0, 0.0
                           ).astype(jnp.bfloat16)

        cm = (jnp.sum(r * r) + jnp.sum(m)) / (T * D)
        counts = jnp.dot(ones_row, onehot,
                         preferred_element_type=jnp.float32)[0]
        p = counts * (1.0 / T)
        klv = jnp.sum(p * jnp.log(p * K + 1e-10))

        codes_ref[i, 0, :] = codes.astype(jnp.int32)
        cm_ref[i, 0, :] = jnp.full((128,), cm, dtype=jnp.float32)
        kl_ref[i, 0, :] = jnp.full((128,), klv, dtype=jnp.float32)
        out_ref[i] = (jnp.dot(d.astype(jnp.bfloat16), Wpost_bf_ref[...],
                              preferred_element_type=jnp.float32)
                      + bpost_ref[...])


def _sc_gather(N, D):
    b_per_w = N // _NW
    mesh = plsc.VectorSubcoreMesh(core_axis_name="c", subcore_axis_name="s")

    @functools.partial(
        pl.kernel, mesh=mesh,
        out_type=jax.ShapeDtypeStruct((N, D), jnp.float32),
        scratch_types=[
            pltpu.VMEM((_CH,), jnp.int32),
            pltpu.VMEM((_CH, D), jnp.float32),
            pltpu.SemaphoreType.DMA,
        ],
    )
    def k(table_hbm, idx_hbm, out_hbm, idx_v, rows_v, sem):
        wid = lax.axis_index("s") * _NC + lax.axis_index("c")
        base = wid * b_per_w
        for j in range(b_per_w // _CH):
            off = base + j * _CH
            pltpu.sync_copy(idx_hbm.at[pl.ds(off, _CH)], idx_v)
            pltpu.async_copy(table_hbm.at[idx_v], rows_v, sem).wait()
            pltpu.sync_copy(rows_v, out_hbm.at[pl.ds(off, _CH)])

    return k


def _body_add(a_ref, b_ref, o_ref):
    o_ref[...] = a_ref[...] + b_ref[...]


def kernel(enc, dec, W_pre, b_pre, codebook, W_post, b_post):
    B, T, D = enc.shape
    K = codebook.shape[0]
    N = B * T
    cbT = codebook.T
    bpre2 = b_pre.reshape(1, D)
    bpost2 = b_post.reshape(1, D)

    out_shapes = (
        jax.ShapeDtypeStruct((B, T, D), jnp.float32),
        jax.ShapeDtypeStruct((B, 1, T), jnp.int32),
        jax.ShapeDtypeStruct((B, 1, 128), jnp.float32),
        jax.ShapeDtypeStruct((B, 1, 128), jnp.float32),
        jax.ShapeDtypeStruct((K, D), jnp.float32),
    )
    full = lambda shape: pl.BlockSpec(shape, lambda b: (0,) * len(shape))
    out_partial, codes3, cm2, kl2, cbW = pl.pallas_call(
        _body_a,
        grid=(B // _SPB,),
        in_specs=[
            pl.BlockSpec((_SPB, T, D), lambda b: (b, 0, 0)),
            pl.BlockSpec((_SPB, T, D), lambda b: (b, 0, 0)),
            full((D, D)),
            full((1, D)),
            full((D, K)),
            full((K, D)),
            full((D, D)),
            full((1, D)),
        ],
        out_specs=(
            pl.BlockSpec((_SPB, T, D), lambda b: (b, 0, 0)),
            pl.BlockSpec((_SPB, 1, T), lambda b: (b, 0, 0)),
            pl.BlockSpec((_SPB, 1, 128), lambda b: (b, 0, 0)),
            pl.BlockSpec((_SPB, 1, 128), lambda b: (b, 0, 0)),
            full((K, D)),
        ),
        scratch_shapes=[
            pltpu.VMEM((K, D), jnp.float32),
            pltpu.VMEM((D, K), jnp.float32),
            pltpu.VMEM((1, K), jnp.float32),
            pltpu.VMEM((D, D), jnp.bfloat16),
        ],
        out_shape=out_shapes,
    )(enc, dec, W_pre, bpre2, cbT, codebook, W_post, bpost2)

    codes_flat = codes3.reshape(N)
    qW = _sc_gather(N, D)(cbW, codes_flat)

    dec_refine = pl.pallas_call(
        _body_add,
        grid=(8,),
        in_specs=[
            pl.BlockSpec((N // 8, D), lambda b: (b, 0)),
            pl.BlockSpec((N // 8, D), lambda b: (b, 0)),
        ],
        out_specs=pl.BlockSpec((N // 8, D), lambda b: (b, 0)),
        out_shape=jax.ShapeDtypeStruct((N, D), jnp.float32),
    )(out_partial.reshape(N, D), qW).reshape(B, T, D)

    cm = cm2[:, 0, 0]
    kl = kl2[:, 0, 0]
    return dec_refine, cm, cm, kl


# speedup vs baseline: 2.3822x; 2.3822x over previous
import jax
import jax.numpy as jnp
from jax.experimental import pallas as pl
from jax.experimental.pallas import tpu as pltpu

_SPB = 4


def _body(enc_ref, dec_ref, Wpre_ref, bpre_ref, cbT_ref, cb_ref, Wpost_ref,
          bpost_ref, out_ref, cm_ref, kl_ref, cbW_ref, cbT2_ref, c2_ref,
          Wpost_bf_ref):
    T, D = enc_ref.shape[1], enc_ref.shape[2]
    K = cb_ref.shape[0]
    b = pl.program_id(0)

    @pl.when(b == 0)
    def _init():
        cb = cb_ref[...]
        Wpost = Wpost_ref[...]
        cbW_ref[...] = jnp.dot(cb, Wpost,
                               preferred_element_type=jnp.float32
                               ).astype(jnp.bfloat16)
        Wpost_bf_ref[...] = Wpost.astype(jnp.bfloat16)
        cbT = cbT_ref[...]
        cbT2_ref[...] = cbT * -2.0
        c2_ref[...] = jnp.sum(cbT * cbT, axis=0, keepdims=True)

    iota = jax.lax.broadcasted_iota(
        jnp.int32, (T, K), 1).astype(jnp.float32)
    ones_row = jnp.ones((8, T), dtype=jnp.bfloat16)
    for i in range(_SPB):
        e = enc_ref[i]
        d = dec_ref[i]
        r = jnp.dot(e - d, Wpre_ref[...],
                    preferred_element_type=jnp.float32) + bpre_ref[...]
        scores = jnp.dot(r, cbT2_ref[...],
                         preferred_element_type=jnp.float32) + c2_ref[...]
        m = jnp.min(scores, axis=1)
        codes = jnp.min(jnp.where(scores <= m[:, None], iota, float(K)),
                        axis=1)
        onehot = jnp.where(iota == codes[:, None], 1.0, 0.0
                           ).astype(jnp.bfloat16)
        qW = jnp.dot(onehot, cbW_ref[...],
                     preferred_element_type=jnp.float32)

        cm = (jnp.sum(r * r) + jnp.sum(m)) / (T * D)
        counts = jnp.dot(ones_row, onehot,
                         preferred_element_type=jnp.float32)[0]
        p = counts * (1.0 / T)
        klv = jnp.sum(p * jnp.log(p * K + 1e-10))

        cm_ref[i, 0, :] = jnp.full((128,), cm, dtype=jnp.float32)
        kl_ref[i, 0, :] = jnp.full((128,), klv, dtype=jnp.float32)
        out_ref[i] = (jnp.dot(d.astype(jnp.bfloat16), Wpost_bf_ref[...],
                              preferred_element_type=jnp.float32)
                      + qW + bpost_ref[...])


def kernel(enc, dec, W_pre, b_pre, codebook, W_post, b_post):
    B, T, D = enc.shape
    K = codebook.shape[0]
    cbT = codebook.T
    bpre2 = b_pre.reshape(1, D)
    bpost2 = b_post.reshape(1, D)

    out_shapes = (
        jax.ShapeDtypeStruct((B, T, D), jnp.float32),
        jax.ShapeDtypeStruct((B, 1, 128), jnp.float32),
        jax.ShapeDtypeStruct((B, 1, 128), jnp.float32),
    )
    full = lambda shape: pl.BlockSpec(shape, lambda b: (0,) * len(shape))
    dec_refine, cm2, kl2 = pl.pallas_call(
        _body,
        grid=(B // _SPB,),
        in_specs=[
            pl.BlockSpec((_SPB, T, D), lambda b: (b, 0, 0)),
            pl.BlockSpec((_SPB, T, D), lambda b: (b, 0, 0)),
            full((D, D)),
            full((1, D)),
            full((D, K)),
            full((K, D)),
            full((D, D)),
            full((1, D)),
        ],
        out_specs=(
            pl.BlockSpec((_SPB, T, D), lambda b: (b, 0, 0)),
            pl.BlockSpec((_SPB, 1, 128), lambda b: (b, 0, 0)),
            pl.BlockSpec((_SPB, 1, 128), lambda b: (b, 0, 0)),
        ),
        scratch_shapes=[
            pltpu.VMEM((K, D), jnp.bfloat16),
            pltpu.VMEM((D, K), jnp.float32),
            pltpu.VMEM((1, K), jnp.float32),
            pltpu.VMEM((D, D), jnp.bfloat16),
        ],
        out_shape=out_shapes,
    )(enc, dec, W_pre, bpre2, cbT, codebook, W_post, bpost2)

    cm = cm2[:, 0, 0]
    kl = kl2[:, 0, 0]
    return dec_refine, cm, cm, kl
